# baseline (speedup 1.0000x reference)
import functools

import jax
import jax.numpy as jnp
from jax.experimental import pallas as pl

N_TOK = 36864
N_CODES = 1024
DIM = 64
BLK = 256


def _split(a):
    hi = a.astype(jnp.bfloat16)
    lo = (a - hi.astype(jnp.float32)).astype(jnp.bfloat16)
    return hi, lo


def _mm(a, b):
    return jax.lax.dot_general(
        a, b, (((1,), (0,)), ((), ())),
        preferred_element_type=jnp.float32)


def _prep_kernel(cbt2_ref, cb_ref, ch_ref, cl_ref, cbh_ref, cbl_ref, c2_ref):
    ch, cl = _split(cbt2_ref[...])
    ch_ref[...] = ch
    cl_ref[...] = cl
    cbh, cbl = _split(cb_ref[...])
    cbh_ref[...] = cbh
    cbl_ref[...] = cbl
    cbt = cbt2_ref[...] * -0.5
    c2_ref[...] = jnp.sum(cbt * cbt, axis=0, keepdims=True)


def _vq_kernel(x_ref, ch_ref, cl_ref, cbh_ref, cbl_ref, c2_ref,
               l2_ref, codes_ref, vec_ref):
    x = x_ref[...]
    xh, xl = _split(x)
    ch = ch_ref[...]
    cross = _mm(xh, ch) + _mm(xh, cl_ref[...]) + _mm(xl, ch)
    e = c2_ref[...] + cross
    x2 = jnp.sum(x * x, axis=1, keepdims=True)
    l2_ref[...] = x2 + e
    codes = jnp.argmin(e, axis=1).astype(jnp.int32)
    codes_ref[...] = codes
    onehot = (codes[:, None] == jax.lax.broadcasted_iota(
        jnp.int32, (1, N_CODES), 1)).astype(jnp.bfloat16)
    vec_ref[...] = _mm(onehot, cbh_ref[...]) + _mm(onehot, cbl_ref[...])


@functools.partial(jax.jit, static_argnames=())
def kernel(x, codebook):
    bf16 = jnp.bfloat16
    f32 = jnp.float32
    cbt2 = -2.0 * codebook.T
    ch, cl, cbh, cbl, c2 = pl.pallas_call(
        _prep_kernel,
        out_shape=[
            jax.ShapeDtypeStruct((DIM, N_CODES), bf16),
            jax.ShapeDtypeStruct((DIM, N_CODES), bf16),
            jax.ShapeDtypeStruct((N_CODES, DIM), bf16),
            jax.ShapeDtypeStruct((N_CODES, DIM), bf16),
            jax.ShapeDtypeStruct((1, N_CODES), f32),
        ],
    )(cbt2, codebook)

    grid = (N_TOK // BLK,)
    l2, codes, vec = pl.pallas_call(
        _vq_kernel,
        grid=grid,
        in_specs=[
            pl.BlockSpec((BLK, DIM), lambda i: (i, 0)),
            pl.BlockSpec((DIM, N_CODES), lambda i: (0, 0)),
            pl.BlockSpec((DIM, N_CODES), lambda i: (0, 0)),
            pl.BlockSpec((N_CODES, DIM), lambda i: (0, 0)),
            pl.BlockSpec((N_CODES, DIM), lambda i: (0, 0)),
            pl.BlockSpec((1, N_CODES), lambda i: (0, 0)),
        ],
        out_specs=[
            pl.BlockSpec((BLK, N_CODES), lambda i: (i, 0)),
            pl.BlockSpec((BLK,), lambda i: (i,)),
            pl.BlockSpec((BLK, DIM), lambda i: (i, 0)),
        ],
        out_shape=[
            jax.ShapeDtypeStruct((N_TOK, N_CODES), f32),
            jax.ShapeDtypeStruct((N_TOK,), jnp.int32),
            jax.ShapeDtypeStruct((N_TOK, DIM), f32),
        ],
    )(x, ch, cl, cbh, cbl, c2)
    return (vec, codes, l2)
